# baseline (speedup 1.0000x reference)
import jax
import jax.numpy as jnp
from jax.experimental import pallas as pl
from jax.experimental.pallas import tpu as pltpu

_LONG, _LAT = 128, 128
_N = _LONG * _LAT
_B = 64
_OUT = 256
_PAD = 256
_OFFSETS = (-1, 1, _LAT, -_LAT, _LAT - 1, _LAT + 1, -_LAT - 1, -_LAT + 1)
_LO, _HI = _LAT + 1, (_LONG - 1) * _LAT - 2
_KC = 1024
_GRID = _N // _KC


def _gnn_kernel(xp_ref, w_ref, b_ref, out_ref, h1_ref):
    k = pl.program_id(0)

    @pl.when(k == 0)
    def _stencil():
        idx = jax.lax.broadcasted_iota(jnp.int32, (1, _N), 1)
        m0 = ((idx >= _LO) & (idx <= _HI)).astype(jnp.float32)
        xv = xp_ref[...]
        h = xv
        for o in _OFFSETS:
            mo = ((idx + o >= _LO) & (idx + o <= _HI)).astype(jnp.float32)
            h = h + (m0 + mo) * pltpu.roll(xv, (-o) % _N, 1)
        h1_ref[...] = h

    hc = h1_ref[:, pl.ds(k * _KC, _KC)]
    acc = jax.lax.dot_general(
        hc, w_ref[...], (((1,), (1,)), ((), ())),
        preferred_element_type=jnp.float32)

    @pl.when(k == 0)
    def _init():
        out_ref[...] = acc + b_ref[...]

    @pl.when(k != 0)
    def _accum():
        out_ref[...] += acc


def kernel(x, W, b):
    xf = x.reshape(_B, _N)
    b2 = b.reshape(1, _OUT)
    return pl.pallas_call(
        _gnn_kernel,
        grid=(_GRID,),
        in_specs=[
            pl.BlockSpec((_B, _N), lambda k: (0, 0)),
            pl.BlockSpec((_OUT, _KC), lambda k: (0, k)),
            pl.BlockSpec((1, _OUT), lambda k: (0, 0)),
        ],
        out_specs=pl.BlockSpec((_B, _OUT), lambda k: (0, 0)),
        out_shape=jax.ShapeDtypeStruct((_B, _OUT), jnp.float32),
        scratch_shapes=[pltpu.VMEM((_B, _N), jnp.float32)],
    )(xf, W, b2)


# speedup vs baseline: 1.3283x; 1.3283x over previous
import jax
import jax.numpy as jnp
from jax.experimental import pallas as pl
from jax.experimental.pallas import tpu as pltpu

_LONG, _LAT = 128, 128
_N = _LONG * _LAT
_B = 64
_OUT = 256
_PAD = 256
_OFFSETS = (-1, 1, _LAT, -_LAT, _LAT - 1, _LAT + 1, -_LAT - 1, -_LAT + 1)
_LO, _HI = _LAT + 1, (_LONG - 1) * _LAT - 2
_KC = 4096
_GRID = _N // _KC


def _gnn_kernel(xp_ref, w_ref, b_ref, out_ref, h1_ref):
    k = pl.program_id(0)

    @pl.when(k == 0)
    def _stencil():
        idx = jax.lax.broadcasted_iota(jnp.int32, (1, _N), 1)
        m0 = ((idx >= _LO) & (idx <= _HI)).astype(jnp.float32)
        xv = xp_ref[...]
        h = xv
        for o in _OFFSETS:
            mo = ((idx + o >= _LO) & (idx + o <= _HI)).astype(jnp.float32)
            h = h + (m0 + mo) * pltpu.roll(xv, (-o) % _N, 1)
        h1_ref[...] = h

    hc = h1_ref[:, pl.ds(k * _KC, _KC)]
    acc = jax.lax.dot_general(
        hc, w_ref[...], (((1,), (1,)), ((), ())),
        preferred_element_type=jnp.float32)

    @pl.when(k == 0)
    def _init():
        out_ref[...] = acc + b_ref[...]

    @pl.when(k != 0)
    def _accum():
        out_ref[...] += acc


def kernel(x, W, b):
    xf = x.reshape(_B, _N)
    b2 = b.reshape(1, _OUT)
    return pl.pallas_call(
        _gnn_kernel,
        grid=(_GRID,),
        in_specs=[
            pl.BlockSpec((_B, _N), lambda k: (0, 0)),
            pl.BlockSpec((_OUT, _KC), lambda k: (0, k)),
            pl.BlockSpec((1, _OUT), lambda k: (0, 0)),
        ],
        out_specs=pl.BlockSpec((_B, _OUT), lambda k: (0, 0)),
        out_shape=jax.ShapeDtypeStruct((_B, _OUT), jnp.float32),
        scratch_shapes=[pltpu.VMEM((_B, _N), jnp.float32)],
    )(xf, W, b2)


# speedup vs baseline: 1.3853x; 1.0429x over previous
import jax
import jax.numpy as jnp
from jax.experimental import pallas as pl
from jax.experimental.pallas import tpu as pltpu

_LONG, _LAT = 128, 128
_N = _LONG * _LAT
_B = 64
_OUT = 256
_PAD = 256
_OFFSETS = (-1, 1, _LAT, -_LAT, _LAT - 1, _LAT + 1, -_LAT - 1, -_LAT + 1)
_LO, _HI = _LAT + 1, (_LONG - 1) * _LAT - 2
_KC = 8192
_GRID = _N // _KC


def _gnn_kernel(xp_ref, w_ref, b_ref, out_ref, h1_ref):
    k = pl.program_id(0)

    @pl.when(k == 0)
    def _stencil():
        idx = jax.lax.broadcasted_iota(jnp.int32, (1, _N), 1)
        m0 = ((idx >= _LO) & (idx <= _HI)).astype(jnp.float32)
        xv = xp_ref[...]
        h = xv
        for o in _OFFSETS:
            mo = ((idx + o >= _LO) & (idx + o <= _HI)).astype(jnp.float32)
            h = h + (m0 + mo) * pltpu.roll(xv, (-o) % _N, 1)
        h1_ref[...] = h

    hc = h1_ref[:, pl.ds(k * _KC, _KC)]
    acc = jax.lax.dot_general(
        hc, w_ref[...], (((1,), (1,)), ((), ())),
        preferred_element_type=jnp.float32)

    @pl.when(k == 0)
    def _init():
        out_ref[...] = acc + b_ref[...]

    @pl.when(k != 0)
    def _accum():
        out_ref[...] += acc


def kernel(x, W, b):
    xf = x.reshape(_B, _N)
    b2 = b.reshape(1, _OUT)
    return pl.pallas_call(
        _gnn_kernel,
        grid=(_GRID,),
        in_specs=[
            pl.BlockSpec((_B, _N), lambda k: (0, 0)),
            pl.BlockSpec((_OUT, _KC), lambda k: (0, k)),
            pl.BlockSpec((1, _OUT), lambda k: (0, 0)),
        ],
        out_specs=pl.BlockSpec((_B, _OUT), lambda k: (0, 0)),
        out_shape=jax.ShapeDtypeStruct((_B, _OUT), jnp.float32),
        scratch_shapes=[pltpu.VMEM((_B, _N), jnp.float32)],
    )(xf, W, b2)


# speedup vs baseline: 1.4387x; 1.0386x over previous
import jax
import jax.numpy as jnp
from jax.experimental import pallas as pl
from jax.experimental.pallas import tpu as pltpu

_LONG, _LAT = 128, 128
_N = _LONG * _LAT
_B = 64
_OUT = 256
_PAD = 256
_OFFSETS = (-1, 1, _LAT, -_LAT, _LAT - 1, _LAT + 1, -_LAT - 1, -_LAT + 1)
_LO, _HI = _LAT + 1, (_LONG - 1) * _LAT - 2
_KC = 16384
_GRID = _N // _KC


def _gnn_kernel(xp_ref, w_ref, b_ref, out_ref, h1_ref):
    k = pl.program_id(0)

    @pl.when(k == 0)
    def _stencil():
        idx = jax.lax.broadcasted_iota(jnp.int32, (1, _N), 1)
        m0 = ((idx >= _LO) & (idx <= _HI)).astype(jnp.float32)
        xv = xp_ref[...]
        h = xv
        for o in _OFFSETS:
            mo = ((idx + o >= _LO) & (idx + o <= _HI)).astype(jnp.float32)
            h = h + (m0 + mo) * pltpu.roll(xv, (-o) % _N, 1)
        h1_ref[...] = h

    hc = h1_ref[:, pl.ds(k * _KC, _KC)]
    acc = jax.lax.dot_general(
        hc, w_ref[...], (((1,), (1,)), ((), ())),
        preferred_element_type=jnp.float32)

    @pl.when(k == 0)
    def _init():
        out_ref[...] = acc + b_ref[...]

    @pl.when(k != 0)
    def _accum():
        out_ref[...] += acc


def kernel(x, W, b):
    xf = x.reshape(_B, _N)
    b2 = b.reshape(1, _OUT)
    return pl.pallas_call(
        _gnn_kernel,
        grid=(_GRID,),
        in_specs=[
            pl.BlockSpec((_B, _N), lambda k: (0, 0)),
            pl.BlockSpec((_OUT, _KC), lambda k: (0, k)),
            pl.BlockSpec((1, _OUT), lambda k: (0, 0)),
        ],
        out_specs=pl.BlockSpec((_B, _OUT), lambda k: (0, 0)),
        out_shape=jax.ShapeDtypeStruct((_B, _OUT), jnp.float32),
        scratch_shapes=[pltpu.VMEM((_B, _N), jnp.float32)],
    )(xf, W, b2)
